# baseline (speedup 1.0000x reference)
import functools

import jax
import jax.numpy as jnp
from jax import lax
from jax.experimental import pallas as pl
from jax.experimental.pallas import tpu as pltpu
from jax.experimental.pallas import tpu_sc as plsc

MU = 0.02
BINS = 10
NC = 2
NS = 16
L = 16
NW = NC * NS

N_TOTAL = 2000000 * 4
PER_W = N_TOTAL // NW
CHUNK = 50000
N_CHUNKS = PER_W // CHUNK
VECS = CHUNK // L
UNROLL = 5
TAB_STRIDE = 256


def _sc_histogram(d_flat):
    mesh = plsc.VectorSubcoreMesh(
        core_axis_name="c", subcore_axis_name="s", num_cores=NC, num_subcores=NS
    )

    @functools.partial(
        pl.kernel,
        out_type=(
            jax.ShapeDtypeStruct((NW, BINS * L), jnp.float32),
            jax.ShapeDtypeStruct((NW, BINS * L), jnp.float32),
        ),
        mesh=mesh,
        compiler_params=pltpu.CompilerParams(needs_layout_passes=False),
        scratch_types=[
            pltpu.VMEM((CHUNK,), jnp.float32),
            pltpu.VMEM((UNROLL * TAB_STRIDE,), jnp.float32),
            pltpu.VMEM((UNROLL * TAB_STRIDE,), jnp.float32),
            pltpu.VMEM((BINS * L,), jnp.float32),
            pltpu.VMEM((BINS * L,), jnp.float32),
        ],
    )
    def hist_kernel(d_hbm, cnt_out, ls_out, d_v, cnt_tab, ls_tab,
                    cnt_fin, ls_fin):
        wid = lax.axis_index("s") * NC + lax.axis_index("c")
        zero16 = jnp.zeros((L,), jnp.float32)
        for k in range(UNROLL):
            for b in range(BINS):
                cnt_tab[pl.ds(k * TAB_STRIDE + b * L, L)] = zero16
                ls_tab[pl.ds(k * TAB_STRIDE + b * L, L)] = zero16

        lane = lax.iota(jnp.int32, L)
        lane_k = [lane + jnp.int32(k * TAB_STRIDE) for k in range(UNROLL)]
        ones16 = jnp.full((L,), 1.0, jnp.float32)
        mu2 = jnp.float32(MU * MU)

        def chunk_body(c, _):
            base = wid * PER_W + c * CHUNK
            pltpu.sync_copy(d_hbm.at[pl.ds(base, CHUNK)], d_v)

            def vec_body(j, _):
                ds = []
                for k in range(UNROLL):
                    off = (j * UNROLL + k) * L
                    ds.append(d_v[pl.ds(off, L)])
                addrs, losses = [], []
                for k in range(UNROLL):
                    d = ds[k]
                    u = d * d
                    v = u + mu2
                    iv = lax.bitcast_convert_type(v, jnp.int32)
                    iv = jnp.int32(0x5F3759DF) - lax.shift_right_arithmetic(
                        iv, jnp.int32(1)
                    )
                    r = lax.bitcast_convert_type(iv, jnp.float32)
                    hv = jnp.float32(0.5) * v
                    r = r * (jnp.float32(1.5) - hv * r * r)
                    r = r * (jnp.float32(1.5) - hv * r * r)
                    s = v * r
                    losses.append(s - jnp.float32(MU))
                    g10 = jnp.abs(d) * r * jnp.float32(BINS)
                    bi = jnp.minimum(g10.astype(jnp.int32), BINS - 1)
                    addrs.append((bi * L) | lane_k[k])
                for k in range(UNROLL):
                    plsc.addupdate_scatter(cnt_tab, [addrs[k]], ones16)
                    plsc.addupdate_scatter(ls_tab, [addrs[k]], losses[k])
                return _

            lax.fori_loop(0, VECS // UNROLL, vec_body, None)
            return _

        lax.fori_loop(0, N_CHUNKS, chunk_body, None)

        for b in range(BINS):
            crow = cnt_tab[pl.ds(b * L, L)]
            lrow = ls_tab[pl.ds(b * L, L)]
            for k in range(1, UNROLL):
                crow += cnt_tab[pl.ds(k * TAB_STRIDE + b * L, L)]
                lrow += ls_tab[pl.ds(k * TAB_STRIDE + b * L, L)]
            cnt_fin[pl.ds(b * L, L)] = crow
            ls_fin[pl.ds(b * L, L)] = lrow
        pltpu.sync_copy(cnt_fin, cnt_out.at[wid])
        pltpu.sync_copy(ls_fin, ls_out.at[wid])

    return hist_kernel(d_flat)


def _combine_kernel(cnt_ref, ls_ref, out_ref):
    n = jnp.float32(0.0)
    acc = jnp.float32(0.0)
    for b in range(BINS):
        cb = jnp.sum(cnt_ref[:, b * L:(b + 1) * L])
        sb = jnp.sum(ls_ref[:, b * L:(b + 1) * L])
        nz = cb > 0
        n += jnp.where(nz, 1.0, 0.0).astype(jnp.float32)
        acc += jnp.where(nz, sb / jnp.maximum(cb, 1.0), 0.0).astype(jnp.float32)
    out_ref[0, 0] = acc / jnp.maximum(n, 1.0)


def kernel(input, target):
    d = (input - target).transpose(1, 0).reshape(-1)
    cnt, ls = _sc_histogram(d)
    res = pl.pallas_call(
        _combine_kernel,
        out_shape=jax.ShapeDtypeStruct((1, 1), jnp.float32),
        out_specs=pl.BlockSpec(memory_space=pltpu.MemorySpace.SMEM),
    )(cnt, ls)
    return res[0, 0]


# speedup vs baseline: 1.0000x; 1.0000x over previous
import functools

import jax
import jax.numpy as jnp
from jax import lax
from jax.experimental import pallas as pl
from jax.experimental.pallas import tpu as pltpu
from jax.experimental.pallas import tpu_sc as plsc

MU = 0.02
BINS = 10
NC = 2
NS = 16
L = 16
NW = NC * NS

N_TOTAL = 2000000 * 4
PER_W = N_TOTAL // NW
CHUNK = 50000
N_CHUNKS = PER_W // CHUNK
VECS = CHUNK // L
UNROLL = 5
TAB_STRIDE = 256


def _sc_histogram(d_flat):
    mesh = plsc.VectorSubcoreMesh(
        core_axis_name="c", subcore_axis_name="s", num_cores=NC, num_subcores=NS
    )

    @functools.partial(
        pl.kernel,
        out_type=(
            jax.ShapeDtypeStruct((NW, BINS * L), jnp.float32),
            jax.ShapeDtypeStruct((NW, BINS * L), jnp.float32),
        ),
        mesh=mesh,
        compiler_params=pltpu.CompilerParams(needs_layout_passes=False),
        scratch_types=[
            pltpu.VMEM((CHUNK,), jnp.float32),
            pltpu.VMEM((UNROLL * TAB_STRIDE,), jnp.float32),
            pltpu.VMEM((UNROLL * TAB_STRIDE,), jnp.float32),
            pltpu.VMEM((BINS * L,), jnp.float32),
            pltpu.VMEM((BINS * L,), jnp.float32),
        ],
    )
    def hist_kernel(d_hbm, cnt_out, ls_out, d_v, cnt_tab, ls_tab,
                    cnt_fin, ls_fin):
        wid = lax.axis_index("s") * NC + lax.axis_index("c")
        zero16 = jnp.zeros((L,), jnp.float32)
        for k in range(UNROLL):
            for b in range(BINS):
                cnt_tab[pl.ds(k * TAB_STRIDE + b * L, L)] = zero16
                ls_tab[pl.ds(k * TAB_STRIDE + b * L, L)] = zero16

        lane = lax.iota(jnp.int32, L)
        lane_k = [lane + jnp.int32(k * TAB_STRIDE) for k in range(UNROLL)]
        ones16 = jnp.full((L,), 1.0, jnp.float32)
        mu2 = jnp.float32(MU * MU)

        def chunk_body(c, _):
            base = wid * PER_W + c * CHUNK
            pltpu.sync_copy(d_hbm.at[pl.ds(base, CHUNK)], d_v)

            def vec_body(j, _):
                ds = []
                for k in range(UNROLL):
                    off = (j * UNROLL + k) * L
                    ds.append(d_v[pl.ds(off, L)])
                addrs, losses = [], []
                for k in range(UNROLL):
                    d = ds[k]
                    u = d * d
                    v = u + mu2
                    iv = lax.bitcast_convert_type(v, jnp.int32)
                    iv = jnp.int32(0x5F3759DF) - lax.shift_right_arithmetic(
                        iv, jnp.int32(1)
                    )
                    r = lax.bitcast_convert_type(iv, jnp.float32)
                    hv = jnp.float32(0.5) * v
                    r = r * (jnp.float32(1.5) - hv * r * r)
                    r = r * (jnp.float32(1.5) - hv * r * r)
                    s = v * r
                    losses.append(s - jnp.float32(MU))
                    g10 = jnp.abs(d) * r * jnp.float32(BINS)
                    bi = jnp.minimum(g10.astype(jnp.int32), BINS - 1)
                    addrs.append((bi * L) | lane_k[k])
                for k in range(UNROLL):
                    plsc.addupdate_scatter(cnt_tab, [addrs[k]], ones16)
                    plsc.addupdate_scatter(ls_tab, [addrs[k]], losses[k])
                return _

            lax.fori_loop(0, VECS // UNROLL, vec_body, None)
            return _

        lax.fori_loop(0, N_CHUNKS, chunk_body, None)

        for b in range(BINS):
            crow = cnt_tab[pl.ds(b * L, L)]
            lrow = ls_tab[pl.ds(b * L, L)]
            for k in range(1, UNROLL):
                crow += cnt_tab[pl.ds(k * TAB_STRIDE + b * L, L)]
                lrow += ls_tab[pl.ds(k * TAB_STRIDE + b * L, L)]
            cnt_fin[pl.ds(b * L, L)] = crow
            ls_fin[pl.ds(b * L, L)] = lrow
        pltpu.sync_copy(cnt_fin, cnt_out.at[wid])
        pltpu.sync_copy(ls_fin, ls_out.at[wid])

    return hist_kernel(d_flat)


def _combine_kernel(cnt_ref, ls_ref, out_ref):
    n = jnp.float32(0.0)
    acc = jnp.float32(0.0)
    for b in range(BINS):
        cb = jnp.sum(cnt_ref[:, b * L:(b + 1) * L])
        sb = jnp.sum(ls_ref[:, b * L:(b + 1) * L])
        nz = cb > 0
        n += jnp.where(nz, 1.0, 0.0).astype(jnp.float32)
        acc += jnp.where(nz, sb / jnp.maximum(cb, 1.0), 0.0).astype(jnp.float32)
    out_ref[0, 0] = acc / jnp.maximum(n, 1.0)


def kernel(input, target):
    d2d = lax.optimization_barrier(input - target)
    d = d2d.transpose(1, 0).reshape(-1)
    cnt, ls = _sc_histogram(d)
    res = pl.pallas_call(
        _combine_kernel,
        out_shape=jax.ShapeDtypeStruct((1, 1), jnp.float32),
        out_specs=pl.BlockSpec(memory_space=pltpu.MemorySpace.SMEM),
    )(cnt, ls)
    return res[0, 0]


# speedup vs baseline: 1.8439x; 1.8438x over previous
import functools

import jax
import jax.numpy as jnp
from jax import lax
from jax.experimental import pallas as pl
from jax.experimental.pallas import tpu as pltpu
from jax.experimental.pallas import tpu_sc as plsc

MU = 0.02
BINS = 10
NC = 2
NS = 16
L = 16
NW = NC * NS

N_TOTAL = 2000000 * 4
PER_W = N_TOTAL // NW
CHUNK = 50000
N_CHUNKS = PER_W // CHUNK
VECS = CHUNK // L
UNROLL = 5
TAB_STRIDE = 256


def _sc_histogram(d_flat):
    mesh = plsc.VectorSubcoreMesh(
        core_axis_name="c", subcore_axis_name="s", num_cores=NC, num_subcores=NS
    )

    @functools.partial(
        pl.kernel,
        out_type=(
            jax.ShapeDtypeStruct((NW, L), jnp.float32),
            jax.ShapeDtypeStruct((NW, L), jnp.float32),
        ),
        mesh=mesh,
        compiler_params=pltpu.CompilerParams(needs_layout_passes=False),
        scratch_types=[
            pltpu.VMEM((CHUNK,), jnp.float32),
            pltpu.VMEM((CHUNK,), jnp.float32),
            pltpu.VMEM((UNROLL * TAB_STRIDE,), jnp.float32),
            pltpu.VMEM((UNROLL * TAB_STRIDE,), jnp.float32),
            pltpu.VMEM((L,), jnp.float32),
            pltpu.VMEM((L,), jnp.float32),
            pltpu.SemaphoreType.DMA,
            pltpu.SemaphoreType.DMA,
        ],
    )
    def hist_kernel(d_hbm, cnt_out, ls_out, d_v0, d_v1, cnt_tab, ls_tab,
                    cnt_fin, ls_fin, sem0, sem1):
        wid = lax.axis_index("s") * NC + lax.axis_index("c")
        zero16 = jnp.zeros((L,), jnp.float32)
        for k in range(UNROLL):
            for sl in range(L):
                cnt_tab[pl.ds(k * TAB_STRIDE + sl * L, L)] = zero16
                ls_tab[pl.ds(k * TAB_STRIDE + sl * L, L)] = zero16

        lane16 = lax.iota(jnp.int32, L) * jnp.int32(L)
        lane_k = [lane16 + jnp.int32(k * TAB_STRIDE) for k in range(UNROLL)]
        ones16 = jnp.full((L,), 1.0, jnp.float32)
        mu2 = jnp.float32(MU * MU)
        base0 = wid * PER_W

        bufs = [d_v0, d_v1]
        sems = [sem0, sem1]

        def dma(c):
            return pltpu.make_async_copy(
                d_hbm.at[pl.ds(base0 + c * CHUNK, CHUNK)],
                bufs[c % 2], sems[c % 2])

        dma(0).start()
        for c in range(N_CHUNKS):
            dma(c).wait()
            if c + 1 < N_CHUNKS:
                dma(c + 1).start()
            d_v = bufs[c % 2]

            def vec_body(j, _):
                ds_ = []
                for k in range(UNROLL):
                    off = (j * UNROLL + k) * L
                    ds_.append(d_v[pl.ds(off, L)])
                addrs, losses = [], []
                for k in range(UNROLL):
                    d = ds_[k]
                    u = d * d
                    v = u + mu2
                    iv = lax.bitcast_convert_type(v, jnp.int32)
                    iv = jnp.int32(0x5F3759DF) - lax.shift_right_arithmetic(
                        iv, jnp.int32(1)
                    )
                    r = lax.bitcast_convert_type(iv, jnp.float32)
                    r = r * (jnp.float32(1.5)
                             - (jnp.float32(0.5) * v) * r * r)
                    s = v * r
                    losses.append(s - jnp.float32(MU))
                    g10 = jnp.abs(d) * r * jnp.float32(BINS)
                    bi = jnp.minimum(g10, jnp.float32(9.5)).astype(jnp.int32)
                    addrs.append(bi + lane_k[k])
                for k in range(UNROLL):
                    plsc.addupdate_scatter(cnt_tab, [addrs[k]], ones16)
                    plsc.addupdate_scatter(ls_tab, [addrs[k]], losses[k])
                return _

            lax.fori_loop(0, VECS // UNROLL, vec_body, None)

        crow = cnt_tab[pl.ds(0, L)]
        lrow = ls_tab[pl.ds(0, L)]
        for k in range(UNROLL):
            for sl in range(L):
                if k == 0 and sl == 0:
                    continue
                off = k * TAB_STRIDE + sl * L
                crow += cnt_tab[pl.ds(off, L)]
                lrow += ls_tab[pl.ds(off, L)]
        cnt_fin[...] = crow
        ls_fin[...] = lrow
        pltpu.sync_copy(cnt_fin, cnt_out.at[wid])
        pltpu.sync_copy(ls_fin, ls_out.at[wid])

    return hist_kernel(d_flat)


def _combine_kernel(cnt_ref, ls_ref, out_ref):
    n = jnp.float32(0.0)
    acc = jnp.float32(0.0)
    for b in range(BINS):
        cb = jnp.sum(cnt_ref[:, b])
        sb = jnp.sum(ls_ref[:, b])
        nz = cb > 0
        n += jnp.where(nz, 1.0, 0.0).astype(jnp.float32)
        acc += jnp.where(nz, sb / jnp.maximum(cb, 1.0), 0.0).astype(jnp.float32)
    out_ref[0, 0] = acc / jnp.maximum(n, 1.0)


def kernel(input, target):
    d = (input - target).reshape(-1, 128, 4).transpose(0, 2, 1).reshape(-1)
    cnt, ls = _sc_histogram(d)
    res = pl.pallas_call(
        _combine_kernel,
        out_shape=jax.ShapeDtypeStruct((1, 1), jnp.float32),
        out_specs=pl.BlockSpec(memory_space=pltpu.MemorySpace.SMEM),
    )(cnt, ls)
    return res[0, 0]


# speedup vs baseline: 2.6092x; 1.4150x over previous
import functools

import jax
import jax.numpy as jnp
from jax import lax
from jax.experimental import pallas as pl
from jax.experimental.pallas import tpu as pltpu
from jax.experimental.pallas import tpu_sc as plsc

MU = 0.02
BINS = 10
NC = 2
NS = 16
L = 16
NW = NC * NS

N_TOTAL = 2000000 * 4
PER_W = N_TOTAL // NW
CHUNK = 50000
N_CHUNKS = PER_W // CHUNK
VECS = CHUNK // L
UNROLL = 5
TAB_STRIDE = 256


def _sc_histogram(d_flat):
    mesh = plsc.VectorSubcoreMesh(
        core_axis_name="c", subcore_axis_name="s", num_cores=NC, num_subcores=NS
    )

    @functools.partial(
        pl.kernel,
        out_type=(
            jax.ShapeDtypeStruct((NW, BINS * L), jnp.float32),
            jax.ShapeDtypeStruct((NW, BINS * L), jnp.float32),
        ),
        mesh=mesh,
        compiler_params=pltpu.CompilerParams(needs_layout_passes=False),
        scratch_types=[
            pltpu.VMEM((CHUNK,), jnp.float32),
            pltpu.VMEM((CHUNK,), jnp.float32),
            pltpu.VMEM((UNROLL * TAB_STRIDE,), jnp.float32),
            pltpu.VMEM((UNROLL * TAB_STRIDE,), jnp.float32),
            pltpu.VMEM((BINS * L,), jnp.float32),
            pltpu.VMEM((BINS * L,), jnp.float32),
            pltpu.SemaphoreType.DMA,
            pltpu.SemaphoreType.DMA,
        ],
    )
    def hist_kernel(d_hbm, cnt_out, ls_out, d_v0, d_v1, cnt_tab, ls_tab,
                    cnt_fin, ls_fin, sem0, sem1):
        wid = lax.axis_index("s") * NC + lax.axis_index("c")
        zero16 = jnp.zeros((L,), jnp.float32)
        for k in range(UNROLL):
            for b in range(BINS):
                cnt_tab[pl.ds(k * TAB_STRIDE + b * L, L)] = zero16
                ls_tab[pl.ds(k * TAB_STRIDE + b * L, L)] = zero16

        lane = lax.iota(jnp.int32, L)
        lane_k = [lane + jnp.int32(k * TAB_STRIDE) for k in range(UNROLL)]
        ones16 = jnp.full((L,), 1.0, jnp.float32)
        mu2 = jnp.float32(MU * MU)
        base0 = wid * PER_W

        bufs = [d_v0, d_v1]
        sems = [sem0, sem1]

        def dma(c):
            return pltpu.make_async_copy(
                d_hbm.at[pl.ds(base0 + c * CHUNK, CHUNK)],
                bufs[c % 2], sems[c % 2])

        dma(0).start()
        for c in range(N_CHUNKS):
            dma(c).wait()
            if c + 1 < N_CHUNKS:
                dma(c + 1).start()
            d_v = bufs[c % 2]

            def vec_body(j, _):
                ds_ = []
                for k in range(UNROLL):
                    off = (j * UNROLL + k) * L
                    ds_.append(d_v[pl.ds(off, L)])
                addrs, losses = [], []
                for k in range(UNROLL):
                    d = ds_[k]
                    u = d * d
                    v = u + mu2
                    iv = lax.bitcast_convert_type(v, jnp.int32)
                    iv = jnp.int32(0x5F3759DF) - lax.shift_right_arithmetic(
                        iv, jnp.int32(1)
                    )
                    r = lax.bitcast_convert_type(iv, jnp.float32)
                    r = r * (jnp.float32(1.5)
                             - (jnp.float32(0.5) * v) * r * r)
                    s = v * r
                    losses.append(s - jnp.float32(MU))
                    g10 = jnp.abs(d) * r * jnp.float32(BINS)
                    bi = jnp.minimum(g10, jnp.float32(9.5)).astype(jnp.int32)
                    addrs.append((bi * L) | lane_k[k])
                for k in range(UNROLL):
                    plsc.addupdate_scatter(cnt_tab, [addrs[k]], ones16)
                    plsc.addupdate_scatter(ls_tab, [addrs[k]], losses[k])
                return _

            lax.fori_loop(0, VECS // UNROLL, vec_body, None)

        for b in range(BINS):
            crow = cnt_tab[pl.ds(b * L, L)]
            lrow = ls_tab[pl.ds(b * L, L)]
            for k in range(1, UNROLL):
                crow += cnt_tab[pl.ds(k * TAB_STRIDE + b * L, L)]
                lrow += ls_tab[pl.ds(k * TAB_STRIDE + b * L, L)]
            cnt_fin[pl.ds(b * L, L)] = crow
            ls_fin[pl.ds(b * L, L)] = lrow
        pltpu.sync_copy(cnt_fin, cnt_out.at[wid])
        pltpu.sync_copy(ls_fin, ls_out.at[wid])

    return hist_kernel(d_flat)


def _combine_kernel(cnt_ref, ls_ref, out_ref):
    n = jnp.float32(0.0)
    acc = jnp.float32(0.0)
    for b in range(BINS):
        cb = jnp.sum(cnt_ref[:, b * L:(b + 1) * L])
        sb = jnp.sum(ls_ref[:, b * L:(b + 1) * L])
        nz = cb > 0
        n += jnp.where(nz, 1.0, 0.0).astype(jnp.float32)
        acc += jnp.where(nz, sb / jnp.maximum(cb, 1.0), 0.0).astype(jnp.float32)
    out_ref[0, 0] = acc / jnp.maximum(n, 1.0)


def kernel(input, target):
    d = (input - target).reshape(-1, 128, 4).transpose(0, 2, 1).reshape(-1)
    cnt, ls = _sc_histogram(d)
    res = pl.pallas_call(
        _combine_kernel,
        out_shape=jax.ShapeDtypeStruct((1, 1), jnp.float32),
        out_specs=pl.BlockSpec(memory_space=pltpu.MemorySpace.SMEM),
    )(cnt, ls)
    return res[0, 0]
